# Initial kernel scaffold; baseline (speedup 1.0000x reference)
import functools

import jax
import jax.numpy as jnp
from jax import lax
from jax.experimental import pallas as pl
from jax.experimental.pallas import tpu as pltpu
from jax.experimental.pallas import tpu_sc as plsc

_NC = 2
_NS = 16
_NW = _NC * _NS
_L = 16


def _make_sc_kernel(B, U, I, D):
    chunk = B // _NW
    mesh = plsc.VectorSubcoreMesh(core_axis_name="c", subcore_axis_name="s")

    @functools.partial(
        pl.kernel,
        out_type=jax.ShapeDtypeStruct((B,), jnp.float32),
        mesh=mesh,
        scratch_types=[
            pltpu.VMEM((U,), jnp.float32),
            pltpu.VMEM((I,), jnp.float32),
            pltpu.VMEM((D,), jnp.float32),
            pltpu.VMEM((chunk,), jnp.int32),
            pltpu.VMEM((chunk,), jnp.int32),
            pltpu.VMEM((chunk,), jnp.int32),
            pltpu.VMEM((chunk,), jnp.float32),
        ],
    )
    def sc_kernel(uidx_hbm, iidx_hbm, didx_hbm, utab_hbm, itab_hbm,
                  dtab_hbm, out_hbm, utab_v, itab_v, dtab_v, uidx_v,
                  iidx_v, didx_v, out_v):
        wid = lax.axis_index("s") * _NC + lax.axis_index("c")
        base = wid * chunk
        pltpu.sync_copy(utab_hbm, utab_v)
        pltpu.sync_copy(itab_hbm, itab_v)
        pltpu.sync_copy(dtab_hbm, dtab_v)
        pltpu.sync_copy(uidx_hbm.at[pl.ds(base, chunk)], uidx_v)
        pltpu.sync_copy(iidx_hbm.at[pl.ds(base, chunk)], iidx_v)
        pltpu.sync_copy(didx_hbm.at[pl.ds(base, chunk)], didx_v)
        for j in range(chunk // _L):
            sl = pl.ds(j * _L, _L)
            u = plsc.load_gather(utab_v, [uidx_v[sl]])
            i = plsc.load_gather(itab_v, [iidx_v[sl]])
            d = plsc.load_gather(dtab_v, [didx_v[sl]])
            out_v[sl] = jnp.exp((u + i + d) / 5.0)
        pltpu.sync_copy(out_v, out_hbm.at[pl.ds(base, chunk)])

    return sc_kernel


@jax.jit
def kernel(x, obs_rew, user_bias, item_bias, data_bias):
    B = x.shape[0]
    u_idx = x[:, 0].astype(jnp.int32)
    i_idx = x[:, 1].astype(jnp.int32)
    d_idx = obs_rew.astype(jnp.int32)
    utab = user_bias.reshape(-1)
    itab = item_bias.reshape(-1)
    dtab = jnp.pad(data_bias.reshape(-1), (0, 8 - data_bias.shape[0]))
    sc = _make_sc_kernel(B, utab.shape[0], itab.shape[0], dtab.shape[0])
    out = sc(u_idx, i_idx, d_idx, utab, itab, dtab)
    return out.reshape(B, 1)


# speedup vs baseline: 4.6885x; 4.6885x over previous
import functools

import jax
import jax.numpy as jnp
from jax import lax
from jax.experimental import pallas as pl
from jax.experimental.pallas import tpu as pltpu
from jax.experimental.pallas import tpu_sc as plsc

_NC = 2
_NS = 16
_NW = _NC * _NS
_L = 16


def _make_sc_kernel(B, U, I, D):
    chunk = B // _NW
    mesh = plsc.VectorSubcoreMesh(core_axis_name="c", subcore_axis_name="s")

    @functools.partial(
        pl.kernel,
        out_type=jax.ShapeDtypeStruct((B,), jnp.float32),
        mesh=mesh,
        compiler_params=pltpu.CompilerParams(needs_layout_passes=False),
        scratch_types=[
            pltpu.VMEM((U,), jnp.float32),
            pltpu.VMEM((I,), jnp.float32),
            pltpu.VMEM((D,), jnp.float32),
            pltpu.VMEM((chunk,), jnp.int32),
            pltpu.VMEM((chunk,), jnp.int32),
            pltpu.VMEM((chunk,), jnp.int32),
            pltpu.VMEM((chunk,), jnp.float32),
        ],
    )
    def sc_kernel(uidx_hbm, iidx_hbm, didx_hbm, utab_hbm, itab_hbm,
                  dtab_hbm, out_hbm, utab_v, itab_v, dtab_v, uidx_v,
                  iidx_v, didx_v, out_v):
        wid = lax.axis_index("s") * _NC + lax.axis_index("c")
        base = wid * chunk
        pltpu.sync_copy(utab_hbm, utab_v)
        pltpu.sync_copy(itab_hbm, itab_v)
        pltpu.sync_copy(dtab_hbm, dtab_v)
        pltpu.sync_copy(uidx_hbm.at[pl.ds(base, chunk)], uidx_v)
        pltpu.sync_copy(iidx_hbm.at[pl.ds(base, chunk)], iidx_v)
        pltpu.sync_copy(didx_hbm.at[pl.ds(base, chunk)], didx_v)
        for j in range(chunk // _L):
            sl = pl.ds(j * _L, _L)
            u = plsc.load_gather(utab_v, [uidx_v[sl]])
            i = plsc.load_gather(itab_v, [iidx_v[sl]])
            d = plsc.load_gather(dtab_v, [didx_v[sl]])
            out_v[sl] = jnp.exp((u + i + d) / 5.0)
        pltpu.sync_copy(out_v, out_hbm.at[pl.ds(base, chunk)])

    return sc_kernel


@jax.jit
def kernel(x, obs_rew, user_bias, item_bias, data_bias):
    B = x.shape[0]
    u_idx = x[:, 0].astype(jnp.int32)
    i_idx = x[:, 1].astype(jnp.int32)
    d_idx = obs_rew.astype(jnp.int32)
    utab = user_bias.reshape(-1)
    itab = item_bias.reshape(-1)
    dtab = jnp.pad(data_bias.reshape(-1), (0, 8 - data_bias.shape[0]))
    sc = _make_sc_kernel(B, utab.shape[0], itab.shape[0], dtab.shape[0])
    out = sc(u_idx, i_idx, d_idx, utab, itab, dtab)
    return out.reshape(B, 1)
